# baseline (device time: 10478 ns/iter reference)
import jax
import jax.numpy as jnp
from jax import lax
from jax.experimental import pallas as pl
from jax.experimental.pallas import tpu as pltpu

N_DEV = 4


def kernel(x):
    m, n = x.shape
    hm = m // 2

    def body(x_ref, out_hbm, e_buf, stats_ref, send_sems, recv_sems, out_sems):
        my = lax.axis_index("i")

        barrier_sem = pltpu.get_barrier_semaphore()
        for r in range(1, N_DEV):
            pl.semaphore_signal(
                barrier_sem, inc=1,
                device_id=((my + r) % N_DEV,),
                device_id_type=pl.DeviceIdType.MESH,
            )

        e = jnp.exp(x_ref[:, :])
        e_buf[:, :] = e
        ls = jnp.sum(e, axis=1, keepdims=True)
        stats_ref[0, :, :] = jnp.transpose(ls, (1, 0))

        pl.semaphore_wait(barrier_sem, N_DEV - 1)

        rdmas = []
        for r in range(1, N_DEV):
            rdma = pltpu.make_async_remote_copy(
                src_ref=stats_ref.at[0],
                dst_ref=stats_ref.at[N_DEV - r],
                send_sem=send_sems.at[r],
                recv_sem=recv_sems.at[N_DEV - r],
                device_id=((my + r) % N_DEV,),
                device_id_type=pl.DeviceIdType.MESH,
            )
            rdma.start()
            rdmas.append(rdma)
        for rdma in rdmas:
            rdma.wait_recv()

        gs = jnp.sum(stats_ref[:, :, :], axis=0)
        scale = jnp.transpose(1.0 / gs, (1, 0))

        out_copies = []
        for h in range(2):
            rows = pl.ds(h * hm, hm)
            e_buf[rows, :] = e_buf[rows, :] * scale[h * hm:(h + 1) * hm, :]
            cp = pltpu.make_async_copy(
                e_buf.at[rows], out_hbm.at[rows], out_sems.at[h],
            )
            cp.start()
            out_copies.append(cp)
        for cp in out_copies:
            cp.wait()
        for rdma in rdmas:
            rdma.wait_send()

    return pl.pallas_call(
        body,
        out_shape=jax.ShapeDtypeStruct((m, n), x.dtype),
        in_specs=[pl.BlockSpec(memory_space=pltpu.VMEM)],
        out_specs=pl.BlockSpec(memory_space=pl.ANY),
        scratch_shapes=[
            pltpu.VMEM((m, n), jnp.float32),
            pltpu.VMEM((N_DEV, 1, m), jnp.float32),
            pltpu.SemaphoreType.DMA((N_DEV,)),
            pltpu.SemaphoreType.DMA((N_DEV,)),
            pltpu.SemaphoreType.DMA((2,)),
        ],
        compiler_params=pltpu.CompilerParams(collective_id=0),
    )(x)


# device time: 9986 ns/iter; 1.0493x vs baseline; 1.0493x over previous
import jax
import jax.numpy as jnp
from jax import lax
from jax.experimental import pallas as pl
from jax.experimental.pallas import tpu as pltpu

N_DEV = 4


def kernel(x):
    m, n = x.shape

    def body(x_ref, out_ref, stats_ref, send_sems, recv_sems):
        my = lax.axis_index("i")

        barrier_sem = pltpu.get_barrier_semaphore()
        for r in range(1, N_DEV):
            pl.semaphore_signal(
                barrier_sem, inc=1,
                device_id=((my + r) % N_DEV,),
                device_id_type=pl.DeviceIdType.MESH,
            )

        e = jnp.exp(x_ref[:, :])
        out_ref[:, :] = e
        stats_ref[0, :, :] = lax.dot_general(
            jnp.ones((1, n), jnp.float32), e,
            (((1,), (1,)), ((), ())),
            preferred_element_type=jnp.float32,
        )

        pl.semaphore_wait(barrier_sem, N_DEV - 1)

        rdmas = []
        for r in range(1, N_DEV):
            rdma = pltpu.make_async_remote_copy(
                src_ref=stats_ref.at[0],
                dst_ref=stats_ref.at[N_DEV - r],
                send_sem=send_sems.at[r],
                recv_sem=recv_sems.at[N_DEV - r],
                device_id=((my + r) % N_DEV,),
                device_id_type=pl.DeviceIdType.MESH,
            )
            rdma.start()
            rdmas.append(rdma)
        for rdma in rdmas:
            rdma.wait_recv()

        gs = jnp.sum(stats_ref[:, :, :], axis=0)
        out_ref[:, :] = out_ref[:, :] * jnp.transpose(1.0 / gs, (1, 0))

        for rdma in rdmas:
            rdma.wait_send()

    return pl.pallas_call(
        body,
        out_shape=jax.ShapeDtypeStruct((m, n), x.dtype),
        in_specs=[pl.BlockSpec(memory_space=pltpu.VMEM)],
        out_specs=pl.BlockSpec(memory_space=pltpu.VMEM),
        scratch_shapes=[
            pltpu.VMEM((N_DEV, 1, m), jnp.float32),
            pltpu.SemaphoreType.DMA((N_DEV,)),
            pltpu.SemaphoreType.DMA((N_DEV,)),
        ],
        compiler_params=pltpu.CompilerParams(collective_id=0),
    )(x)
